# baseline (speedup 1.0000x reference)
import functools

import jax
import jax.numpy as jnp
from jax import lax
from jax.experimental import pallas as pl
from jax.experimental.pallas import tpu as pltpu
from jax.experimental.pallas import tpu_sc as plsc

N_SEG = 64
D = 128
L = 16
NC = 2
NS = 16
NW = NC * NS
BLOCK = 400
NBUF = 2
N_CHUNK = D // L


SC_ROWS = 51200
RB_TC = 800


def _partial_body(
    nf_hbm, ids_hbm, pa_hbm, pc_hbm,
    rows0_v, rows1_v, ids0_v, ids1_v,
    acc_v, cnt_v, sem0, sem1,
):
    n_blocks = SC_ROWS // BLOCK
    max_k = (n_blocks + NW - 1) // NW
    n_rounds = (max_k + NBUF - 1) // NBUF

    cid = lax.axis_index("c")
    sid = lax.axis_index("s")
    wid = sid * NC + cid

    zero16 = jnp.zeros((L,), jnp.float32)
    one16 = jnp.ones((L,), jnp.float32)
    sems = (sem0, sem1)
    rows_bufs = (rows0_v, rows1_v)
    ids_bufs = (ids0_v, ids1_v)

    def _zero_acc(i, carry):
        s = i // N_CHUNK
        j = i % N_CHUNK
        acc_v[s, pl.ds(j * L, L)] = zero16
        return carry

    lax.fori_loop(0, N_SEG * N_CHUNK, _zero_acc, 0)

    def _zero_cnt(s, carry):
        cnt_v[s, :] = zero16
        return carry

    lax.fori_loop(0, N_SEG, _zero_cnt, 0)

    def _issue(k, ph):
        blk = wid + k * NW

        @pl.when(blk < n_blocks)
        def _():
            base = blk * BLOCK
            pltpu.async_copy(
                nf_hbm.at[pl.ds(base, BLOCK), :], rows_bufs[ph], sems[ph]
            )
            pltpu.async_copy(ids_hbm.at[pl.ds(base, BLOCK)], ids_bufs[ph], sems[ph])

    def _wait(ph):
        pltpu.make_async_copy(
            nf_hbm.at[pl.ds(0, BLOCK), :], rows_bufs[ph], sems[ph]
        ).wait()
        pltpu.make_async_copy(
            ids_hbm.at[pl.ds(0, BLOCK)], ids_bufs[ph], sems[ph]
        ).wait()

    def _tree16(rows_v, r0, sl):
        s0 = rows_v[r0 + 0, sl] + rows_v[r0 + 1, sl]
        s1 = rows_v[r0 + 2, sl] + rows_v[r0 + 3, sl]
        s2 = rows_v[r0 + 4, sl] + rows_v[r0 + 5, sl]
        s3 = rows_v[r0 + 6, sl] + rows_v[r0 + 7, sl]
        s4 = rows_v[r0 + 8, sl] + rows_v[r0 + 9, sl]
        s5 = rows_v[r0 + 10, sl] + rows_v[r0 + 11, sl]
        s6 = rows_v[r0 + 12, sl] + rows_v[r0 + 13, sl]
        s7 = rows_v[r0 + 14, sl] + rows_v[r0 + 15, sl]
        t0 = s0 + s1
        t1 = s2 + s3
        t2 = s4 + s5
        t3 = s6 + s7
        return (t0 + t1) + (t2 + t3)

    def _process(ph):
        rows_v = rows_bufs[ph]
        ids_v = ids_bufs[ph]

        def _group(g, carry):
            r0 = g * L
            idvec = ids_v[pl.ds(r0, L)]
            s_first = idvec[0]
            s_last = idvec[L - 1]

            @pl.when(s_first == s_last)
            def _uniform():
                for j in range(N_CHUNK):
                    sl = pl.ds(j * L, L)
                    total = _tree16(rows_v, r0, sl)
                    acc_v[s_first, sl] = acc_v[s_first, sl] + total
                cnt_v[s_first, :] = cnt_v[s_first, :] + (one16 * 16.0)

            @pl.when(s_first != s_last)
            def _mixed():
                for r in range(L):
                    seg = idvec[r]
                    for j in range(N_CHUNK):
                        sl = pl.ds(j * L, L)
                        acc_v[seg, sl] = acc_v[seg, sl] + rows_v[r0 + r, sl]
                    cnt_v[seg, :] = cnt_v[seg, :] + one16

            return carry

        ida = ids_v[pl.ds(0, L)]
        idb = ids_v[pl.ds(BLOCK - L, L)]
        b_first = ida[0]
        b_last = idb[L - 1]

        @pl.when(b_first == b_last)
        def _block_uniform():
            def _gsum(g, run):
                r0 = g * L
                return tuple(
                    run[j] + _tree16(rows_v, r0, pl.ds(j * L, L))
                    for j in range(N_CHUNK)
                )

            run = lax.fori_loop(
                0, BLOCK // L, _gsum, tuple(zero16 for _ in range(N_CHUNK))
            )
            for j in range(N_CHUNK):
                sl = pl.ds(j * L, L)
                acc_v[b_first, sl] = acc_v[b_first, sl] + run[j]
            cnt_v[b_first, :] = cnt_v[b_first, :] + (one16 * float(BLOCK))

        @pl.when(b_first != b_last)
        def _block_mixed():
            lax.fori_loop(0, BLOCK // L, _group, 0)

    for ph in range(NBUF):
        _issue(ph, ph)

    def _round(kp, carry):
        for ph in range(NBUF):
            k = kp * NBUF + ph
            blk = wid + k * NW

            @pl.when(blk < n_blocks)
            def _():
                _wait(ph)
                _process(ph)
                _issue(k + NBUF, ph)

        return carry

    lax.fori_loop(0, n_rounds, _round, 0)

    pltpu.sync_copy(acc_v, pa_hbm.at[:, wid, :])
    pltpu.sync_copy(cnt_v, pc_hbm.at[:, wid, :])


def _partial_tc_body(ids_ref, rows_ref, sum_ref, cnt_ref):
    i = pl.program_id(0)
    ids2d = ids_ref[0]
    iota_seg = lax.broadcasted_iota(jnp.int32, (N_SEG, RB_TC), 0)
    a = (iota_seg == jnp.broadcast_to(ids2d, (N_SEG, RB_TC))).astype(jnp.float32)
    psum = jnp.dot(a, rows_ref[...], preferred_element_type=jnp.float32)
    pcnt = jnp.broadcast_to(jnp.sum(a, axis=1)[:, None], (N_SEG, D))

    @pl.when(i == 0)
    def _init():
        sum_ref[...] = psum
        cnt_ref[...] = pcnt

    @pl.when(i != 0)
    def _acc():
        sum_ref[...] += psum
        cnt_ref[...] += pcnt


def _merge_tc_body(pa_ref, pc_ref, tsum_ref, tcnt_ref, out_ref):
    sums = jnp.sum(pa_ref[...], axis=1) + tsum_ref[...]
    cnts = jnp.sum(pc_ref[...], axis=1)[:, 0:1] + tcnt_ref[:, 0:1]
    denom = jnp.maximum(cnts, 1.0)
    out_ref[...] = sums / denom


@jax.jit
def kernel(node_features, batch):
    mesh = plsc.VectorSubcoreMesh(
        core_axis_name="c", subcore_axis_name="s", num_cores=NC, num_subcores=NS
    )

    partial_fn = pl.kernel(
        _partial_body,
        out_type=(
            jax.ShapeDtypeStruct((N_SEG, NW, D), jnp.float32),
            jax.ShapeDtypeStruct((N_SEG, NW, L), jnp.float32),
        ),
        mesh=mesh,
        scratch_types=(
            pltpu.VMEM((BLOCK, D), jnp.float32),
            pltpu.VMEM((BLOCK, D), jnp.float32),
            pltpu.VMEM((BLOCK,), jnp.int32),
            pltpu.VMEM((BLOCK,), jnp.int32),
            pltpu.VMEM((N_SEG, D), jnp.float32),
            pltpu.VMEM((N_SEG, L), jnp.float32),
            pltpu.SemaphoreType.DMA,
            pltpu.SemaphoreType.DMA,
        ),
    )
    pa, pc = partial_fn(node_features, batch)

    n_rows = node_features.shape[0]
    n_tc = n_rows - SC_ROWS
    n_tb = n_tc // RB_TC
    tb0 = SC_ROWS // RB_TC
    ids_tc = batch.reshape(n_rows // RB_TC, 1, RB_TC)
    rows_tc = node_features
    tsum, tcnt = pl.pallas_call(
        _partial_tc_body,
        grid=(n_tb,),
        in_specs=[
            pl.BlockSpec((1, 1, RB_TC), lambda i: (tb0 + i, 0, 0)),
            pl.BlockSpec((RB_TC, D), lambda i: (tb0 + i, 0)),
        ],
        out_specs=[
            pl.BlockSpec((N_SEG, D), lambda i: (0, 0)),
            pl.BlockSpec((N_SEG, D), lambda i: (0, 0)),
        ],
        out_shape=(
            jax.ShapeDtypeStruct((N_SEG, D), jnp.float32),
            jax.ShapeDtypeStruct((N_SEG, D), jnp.float32),
        ),
    )(ids_tc, rows_tc)

    return pl.pallas_call(
        _merge_tc_body,
        out_shape=jax.ShapeDtypeStruct((N_SEG, D), jnp.float32),
    )(pa, pc, tsum, tcnt)


# speedup vs baseline: 1.2194x; 1.2194x over previous
import functools

import jax
import jax.numpy as jnp
from jax import lax
from jax.experimental import pallas as pl
from jax.experimental.pallas import tpu as pltpu
from jax.experimental.pallas import tpu_sc as plsc

N_SEG = 64
D = 128
L = 16
NC = 2
NS = 16
NW = NC * NS
BLOCK = 400
NBUF = 2
N_CHUNK = D // L


SC_ROWS = 64000
RB_TC = 2000


def _partial_body(
    nf_hbm, ids_hbm, pa_hbm, pc_hbm,
    rows0_v, rows1_v, ids0_v, ids1_v,
    acc_v, cnt_v, sem0, sem1,
):
    n_blocks = SC_ROWS // BLOCK
    max_k = (n_blocks + NW - 1) // NW
    n_rounds = (max_k + NBUF - 1) // NBUF

    cid = lax.axis_index("c")
    sid = lax.axis_index("s")
    wid = sid * NC + cid

    zero16 = jnp.zeros((L,), jnp.float32)
    one16 = jnp.ones((L,), jnp.float32)
    sems = (sem0, sem1)
    rows_bufs = (rows0_v, rows1_v)
    ids_bufs = (ids0_v, ids1_v)

    def _zero_acc(i, carry):
        s = i // N_CHUNK
        j = i % N_CHUNK
        acc_v[s, pl.ds(j * L, L)] = zero16
        return carry

    lax.fori_loop(0, N_SEG * N_CHUNK, _zero_acc, 0)

    def _zero_cnt(s, carry):
        cnt_v[s, :] = zero16
        return carry

    lax.fori_loop(0, N_SEG, _zero_cnt, 0)

    def _issue(k, ph):
        blk = wid + k * NW

        @pl.when(blk < n_blocks)
        def _():
            base = blk * BLOCK
            pltpu.async_copy(
                nf_hbm.at[pl.ds(base, BLOCK), :], rows_bufs[ph], sems[ph]
            )
            pltpu.async_copy(ids_hbm.at[pl.ds(base, BLOCK)], ids_bufs[ph], sems[ph])

    def _wait(ph):
        pltpu.make_async_copy(
            nf_hbm.at[pl.ds(0, BLOCK), :], rows_bufs[ph], sems[ph]
        ).wait()
        pltpu.make_async_copy(
            ids_hbm.at[pl.ds(0, BLOCK)], ids_bufs[ph], sems[ph]
        ).wait()

    def _tree16(rows_v, r0, sl):
        s0 = rows_v[r0 + 0, sl] + rows_v[r0 + 1, sl]
        s1 = rows_v[r0 + 2, sl] + rows_v[r0 + 3, sl]
        s2 = rows_v[r0 + 4, sl] + rows_v[r0 + 5, sl]
        s3 = rows_v[r0 + 6, sl] + rows_v[r0 + 7, sl]
        s4 = rows_v[r0 + 8, sl] + rows_v[r0 + 9, sl]
        s5 = rows_v[r0 + 10, sl] + rows_v[r0 + 11, sl]
        s6 = rows_v[r0 + 12, sl] + rows_v[r0 + 13, sl]
        s7 = rows_v[r0 + 14, sl] + rows_v[r0 + 15, sl]
        t0 = s0 + s1
        t1 = s2 + s3
        t2 = s4 + s5
        t3 = s6 + s7
        return (t0 + t1) + (t2 + t3)

    def _process(ph):
        rows_v = rows_bufs[ph]
        ids_v = ids_bufs[ph]

        def _group(g, carry):
            r0 = g * L
            idvec = ids_v[pl.ds(r0, L)]
            s_first = idvec[0]
            s_last = idvec[L - 1]

            @pl.when(s_first == s_last)
            def _uniform():
                for j in range(N_CHUNK):
                    sl = pl.ds(j * L, L)
                    total = _tree16(rows_v, r0, sl)
                    acc_v[s_first, sl] = acc_v[s_first, sl] + total
                cnt_v[s_first, :] = cnt_v[s_first, :] + (one16 * 16.0)

            @pl.when(s_first != s_last)
            def _mixed():
                for r in range(L):
                    seg = idvec[r]
                    for j in range(N_CHUNK):
                        sl = pl.ds(j * L, L)
                        acc_v[seg, sl] = acc_v[seg, sl] + rows_v[r0 + r, sl]
                    cnt_v[seg, :] = cnt_v[seg, :] + one16

            return carry

        ida = ids_v[pl.ds(0, L)]
        idb = ids_v[pl.ds(BLOCK - L, L)]
        b_first = ida[0]
        b_last = idb[L - 1]

        @pl.when(b_first == b_last)
        def _block_uniform():
            def _gsum(g, run):
                r0 = g * L
                return tuple(
                    run[j] + _tree16(rows_v, r0, pl.ds(j * L, L))
                    for j in range(N_CHUNK)
                )

            run = lax.fori_loop(
                0, BLOCK // L, _gsum, tuple(zero16 for _ in range(N_CHUNK))
            )
            for j in range(N_CHUNK):
                sl = pl.ds(j * L, L)
                acc_v[b_first, sl] = acc_v[b_first, sl] + run[j]
            cnt_v[b_first, :] = cnt_v[b_first, :] + (one16 * float(BLOCK))

        @pl.when(b_first != b_last)
        def _block_mixed():
            lax.fori_loop(0, BLOCK // L, _group, 0)

    for ph in range(NBUF):
        _issue(ph, ph)

    def _round(kp, carry):
        for ph in range(NBUF):
            k = kp * NBUF + ph
            blk = wid + k * NW

            @pl.when(blk < n_blocks)
            def _():
                _wait(ph)
                _process(ph)
                _issue(k + NBUF, ph)

        return carry

    lax.fori_loop(0, n_rounds, _round, 0)

    pltpu.sync_copy(acc_v, pa_hbm.at[:, wid, :])
    pltpu.sync_copy(cnt_v, pc_hbm.at[:, wid, :])


def _partial_tc_body(ids_ref, rows_ref, sum_ref, cnt_ref):
    i = pl.program_id(0)
    ids2d = ids_ref[0]
    iota_seg = lax.broadcasted_iota(jnp.int32, (N_SEG, RB_TC), 0)
    a = (iota_seg == jnp.broadcast_to(ids2d, (N_SEG, RB_TC))).astype(jnp.float32)
    psum = jnp.dot(a, rows_ref[...], preferred_element_type=jnp.float32)
    pcnt = jnp.broadcast_to(jnp.sum(a, axis=1)[:, None], (N_SEG, D))

    @pl.when(i == 0)
    def _init():
        sum_ref[...] = psum
        cnt_ref[...] = pcnt

    @pl.when(i != 0)
    def _acc():
        sum_ref[...] += psum
        cnt_ref[...] += pcnt


def _merge_tc_body(pa_ref, pc_ref, tsum_ref, tcnt_ref, out_ref):
    sums = jnp.sum(pa_ref[...], axis=1) + tsum_ref[...]
    cnts = jnp.sum(pc_ref[...], axis=1)[:, 0:1] + tcnt_ref[:, 0:1]
    denom = jnp.maximum(cnts, 1.0)
    out_ref[...] = sums / denom


@jax.jit
def kernel(node_features, batch):
    mesh = plsc.VectorSubcoreMesh(
        core_axis_name="c", subcore_axis_name="s", num_cores=NC, num_subcores=NS
    )

    partial_fn = pl.kernel(
        _partial_body,
        out_type=(
            jax.ShapeDtypeStruct((N_SEG, NW, D), jnp.float32),
            jax.ShapeDtypeStruct((N_SEG, NW, L), jnp.float32),
        ),
        mesh=mesh,
        scratch_types=(
            pltpu.VMEM((BLOCK, D), jnp.float32),
            pltpu.VMEM((BLOCK, D), jnp.float32),
            pltpu.VMEM((BLOCK,), jnp.int32),
            pltpu.VMEM((BLOCK,), jnp.int32),
            pltpu.VMEM((N_SEG, D), jnp.float32),
            pltpu.VMEM((N_SEG, L), jnp.float32),
            pltpu.SemaphoreType.DMA,
            pltpu.SemaphoreType.DMA,
        ),
    )
    pa, pc = partial_fn(node_features, batch)

    n_rows = node_features.shape[0]
    n_tc = n_rows - SC_ROWS
    n_tb = n_tc // RB_TC
    tb0 = SC_ROWS // RB_TC
    ids_tc = batch.reshape(n_rows // RB_TC, 1, RB_TC)
    rows_tc = node_features
    tsum, tcnt = pl.pallas_call(
        _partial_tc_body,
        grid=(n_tb,),
        in_specs=[
            pl.BlockSpec((1, 1, RB_TC), lambda i: (tb0 + i, 0, 0)),
            pl.BlockSpec((RB_TC, D), lambda i: (tb0 + i, 0)),
        ],
        out_specs=[
            pl.BlockSpec((N_SEG, D), lambda i: (0, 0)),
            pl.BlockSpec((N_SEG, D), lambda i: (0, 0)),
        ],
        out_shape=(
            jax.ShapeDtypeStruct((N_SEG, D), jnp.float32),
            jax.ShapeDtypeStruct((N_SEG, D), jnp.float32),
        ),
    )(ids_tc, rows_tc)

    return pl.pallas_call(
        _merge_tc_body,
        out_shape=jax.ShapeDtypeStruct((N_SEG, D), jnp.float32),
    )(pa, pc, tsum, tcnt)


# speedup vs baseline: 1.2883x; 1.0565x over previous
import functools

import jax
import jax.numpy as jnp
from jax import lax
from jax.experimental import pallas as pl
from jax.experimental.pallas import tpu as pltpu
from jax.experimental.pallas import tpu_sc as plsc

N_SEG = 64
D = 128
L = 16
NC = 2
NS = 16
NW = NC * NS
BLOCK = 400
NBUF = 2
N_CHUNK = D // L


SC_ROWS = 50000
RB_TC = 2000


def _partial_body(
    nf_hbm, ids_hbm, pa_hbm, pc_hbm,
    rows0_v, rows1_v, ids0_v, ids1_v,
    acc_v, cnt_v, sem0, sem1,
):
    n_blocks = SC_ROWS // BLOCK
    max_k = (n_blocks + NW - 1) // NW
    n_rounds = (max_k + NBUF - 1) // NBUF

    cid = lax.axis_index("c")
    sid = lax.axis_index("s")
    wid = sid * NC + cid

    zero16 = jnp.zeros((L,), jnp.float32)
    one16 = jnp.ones((L,), jnp.float32)
    sems = (sem0, sem1)
    rows_bufs = (rows0_v, rows1_v)
    ids_bufs = (ids0_v, ids1_v)

    def _zero_acc(i, carry):
        s = i // N_CHUNK
        j = i % N_CHUNK
        acc_v[s, pl.ds(j * L, L)] = zero16
        return carry

    lax.fori_loop(0, N_SEG * N_CHUNK, _zero_acc, 0)

    def _zero_cnt(s, carry):
        cnt_v[s, :] = zero16
        return carry

    lax.fori_loop(0, N_SEG, _zero_cnt, 0)

    def _issue(k, ph):
        blk = wid + k * NW

        @pl.when(blk < n_blocks)
        def _():
            base = blk * BLOCK
            pltpu.async_copy(
                nf_hbm.at[pl.ds(base, BLOCK), :], rows_bufs[ph], sems[ph]
            )
            pltpu.async_copy(ids_hbm.at[pl.ds(base, BLOCK)], ids_bufs[ph], sems[ph])

    def _wait(ph):
        pltpu.make_async_copy(
            nf_hbm.at[pl.ds(0, BLOCK), :], rows_bufs[ph], sems[ph]
        ).wait()
        pltpu.make_async_copy(
            ids_hbm.at[pl.ds(0, BLOCK)], ids_bufs[ph], sems[ph]
        ).wait()

    def _tree16(rows_v, r0, sl):
        s0 = rows_v[r0 + 0, sl] + rows_v[r0 + 1, sl]
        s1 = rows_v[r0 + 2, sl] + rows_v[r0 + 3, sl]
        s2 = rows_v[r0 + 4, sl] + rows_v[r0 + 5, sl]
        s3 = rows_v[r0 + 6, sl] + rows_v[r0 + 7, sl]
        s4 = rows_v[r0 + 8, sl] + rows_v[r0 + 9, sl]
        s5 = rows_v[r0 + 10, sl] + rows_v[r0 + 11, sl]
        s6 = rows_v[r0 + 12, sl] + rows_v[r0 + 13, sl]
        s7 = rows_v[r0 + 14, sl] + rows_v[r0 + 15, sl]
        t0 = s0 + s1
        t1 = s2 + s3
        t2 = s4 + s5
        t3 = s6 + s7
        return (t0 + t1) + (t2 + t3)

    def _process(ph):
        rows_v = rows_bufs[ph]
        ids_v = ids_bufs[ph]

        def _group(g, carry):
            r0 = g * L
            idvec = ids_v[pl.ds(r0, L)]
            s_first = idvec[0]
            s_last = idvec[L - 1]

            @pl.when(s_first == s_last)
            def _uniform():
                for j in range(N_CHUNK):
                    sl = pl.ds(j * L, L)
                    total = _tree16(rows_v, r0, sl)
                    acc_v[s_first, sl] = acc_v[s_first, sl] + total
                cnt_v[s_first, :] = cnt_v[s_first, :] + (one16 * 16.0)

            @pl.when(s_first != s_last)
            def _mixed():
                for r in range(L):
                    seg = idvec[r]
                    for j in range(N_CHUNK):
                        sl = pl.ds(j * L, L)
                        acc_v[seg, sl] = acc_v[seg, sl] + rows_v[r0 + r, sl]
                    cnt_v[seg, :] = cnt_v[seg, :] + one16

            return carry

        ida = ids_v[pl.ds(0, L)]
        idb = ids_v[pl.ds(BLOCK - L, L)]
        b_first = ida[0]
        b_last = idb[L - 1]

        @pl.when(b_first == b_last)
        def _block_uniform():
            def _gsum(g, run):
                r0 = g * L
                return tuple(
                    run[j] + _tree16(rows_v, r0, pl.ds(j * L, L))
                    for j in range(N_CHUNK)
                )

            run = lax.fori_loop(
                0, BLOCK // L, _gsum, tuple(zero16 for _ in range(N_CHUNK))
            )
            for j in range(N_CHUNK):
                sl = pl.ds(j * L, L)
                acc_v[b_first, sl] = acc_v[b_first, sl] + run[j]
            cnt_v[b_first, :] = cnt_v[b_first, :] + (one16 * float(BLOCK))

        @pl.when(b_first != b_last)
        def _block_mixed():
            lax.fori_loop(0, BLOCK // L, _group, 0)

    for ph in range(NBUF):
        _issue(ph, ph)

    def _round(kp, carry):
        for ph in range(NBUF):
            k = kp * NBUF + ph
            blk = wid + k * NW

            @pl.when(blk < n_blocks)
            def _():
                _wait(ph)
                _process(ph)
                _issue(k + NBUF, ph)

        return carry

    lax.fori_loop(0, n_rounds, _round, 0)

    pltpu.sync_copy(acc_v, pa_hbm.at[:, wid, :])
    pltpu.sync_copy(cnt_v, pc_hbm.at[:, wid, :])


def _partial_tc_body(ids_ref, rows_ref, sum_ref, cnt_ref):
    i = pl.program_id(0)
    ids2d = ids_ref[0]
    iota_seg = lax.broadcasted_iota(jnp.int32, (N_SEG, RB_TC), 0)
    a = (iota_seg == jnp.broadcast_to(ids2d, (N_SEG, RB_TC))).astype(jnp.float32)
    psum = jnp.dot(a, rows_ref[...], preferred_element_type=jnp.float32)
    pcnt = jnp.broadcast_to(jnp.sum(a, axis=1)[:, None], (N_SEG, D))

    @pl.when(i == 0)
    def _init():
        sum_ref[...] = psum
        cnt_ref[...] = pcnt

    @pl.when(i != 0)
    def _acc():
        sum_ref[...] += psum
        cnt_ref[...] += pcnt


def _merge_tc_body(pa_ref, pc_ref, tsum_ref, tcnt_ref, out_ref):
    sums = jnp.sum(pa_ref[...], axis=1) + tsum_ref[...]
    cnts = jnp.sum(pc_ref[...], axis=1)[:, 0:1] + tcnt_ref[:, 0:1]
    denom = jnp.maximum(cnts, 1.0)
    out_ref[...] = sums / denom


@jax.jit
def kernel(node_features, batch):
    mesh = plsc.VectorSubcoreMesh(
        core_axis_name="c", subcore_axis_name="s", num_cores=NC, num_subcores=NS
    )

    partial_fn = pl.kernel(
        _partial_body,
        out_type=(
            jax.ShapeDtypeStruct((N_SEG, NW, D), jnp.float32),
            jax.ShapeDtypeStruct((N_SEG, NW, L), jnp.float32),
        ),
        mesh=mesh,
        scratch_types=(
            pltpu.VMEM((BLOCK, D), jnp.float32),
            pltpu.VMEM((BLOCK, D), jnp.float32),
            pltpu.VMEM((BLOCK,), jnp.int32),
            pltpu.VMEM((BLOCK,), jnp.int32),
            pltpu.VMEM((N_SEG, D), jnp.float32),
            pltpu.VMEM((N_SEG, L), jnp.float32),
            pltpu.SemaphoreType.DMA,
            pltpu.SemaphoreType.DMA,
        ),
    )
    pa, pc = partial_fn(node_features, batch)

    n_rows = node_features.shape[0]
    n_tc = n_rows - SC_ROWS
    n_tb = n_tc // RB_TC
    tb0 = SC_ROWS // RB_TC
    ids_tc = batch.reshape(n_rows // RB_TC, 1, RB_TC)
    rows_tc = node_features
    tsum, tcnt = pl.pallas_call(
        _partial_tc_body,
        grid=(n_tb,),
        in_specs=[
            pl.BlockSpec((1, 1, RB_TC), lambda i: (tb0 + i, 0, 0)),
            pl.BlockSpec((RB_TC, D), lambda i: (tb0 + i, 0)),
        ],
        out_specs=[
            pl.BlockSpec((N_SEG, D), lambda i: (0, 0)),
            pl.BlockSpec((N_SEG, D), lambda i: (0, 0)),
        ],
        out_shape=(
            jax.ShapeDtypeStruct((N_SEG, D), jnp.float32),
            jax.ShapeDtypeStruct((N_SEG, D), jnp.float32),
        ),
    )(ids_tc, rows_tc)

    return pl.pallas_call(
        _merge_tc_body,
        out_shape=jax.ShapeDtypeStruct((N_SEG, D), jnp.float32),
    )(pa, pc, tsum, tcnt)


# speedup vs baseline: 1.3442x; 1.0434x over previous
import functools

import jax
import jax.numpy as jnp
from jax import lax
from jax.experimental import pallas as pl
from jax.experimental.pallas import tpu as pltpu
from jax.experimental.pallas import tpu_sc as plsc

N_SEG = 64
D = 128
L = 16
NC = 2
NS = 16
NW = NC * NS
BLOCK = 400
NBUF = 2
N_CHUNK = D // L


SC_ROWS = 50000
RB_TC = 2000


def _partial_body(
    nf_hbm, ids_hbm, pa_hbm, pc_hbm,
    rows0_v, rows1_v, ids0_v, ids1_v,
    acc_v, cnt_v, sem0, sem1,
):
    n_blocks = SC_ROWS // BLOCK
    max_k = (n_blocks + NW - 1) // NW
    n_rounds = (max_k + NBUF - 1) // NBUF

    cid = lax.axis_index("c")
    sid = lax.axis_index("s")
    wid = sid * NC + cid

    zero16 = jnp.zeros((L,), jnp.float32)
    one16 = jnp.ones((L,), jnp.float32)
    sems = (sem0, sem1)
    rows_bufs = (rows0_v, rows1_v)
    ids_bufs = (ids0_v, ids1_v)

    def _issue(k, ph):
        blk = wid + k * NW

        @pl.when(blk < n_blocks)
        def _():
            base = blk * BLOCK
            pltpu.async_copy(
                nf_hbm.at[pl.ds(base, BLOCK), :], rows_bufs[ph], sems[ph]
            )
            pltpu.async_copy(ids_hbm.at[pl.ds(base, BLOCK)], ids_bufs[ph], sems[ph])

    def _wait(ph):
        pltpu.make_async_copy(
            nf_hbm.at[pl.ds(0, BLOCK), :], rows_bufs[ph], sems[ph]
        ).wait()
        pltpu.make_async_copy(
            ids_hbm.at[pl.ds(0, BLOCK)], ids_bufs[ph], sems[ph]
        ).wait()

    def _tree16(rows_v, r0, sl):
        s0 = rows_v[r0 + 0, sl] + rows_v[r0 + 1, sl]
        s1 = rows_v[r0 + 2, sl] + rows_v[r0 + 3, sl]
        s2 = rows_v[r0 + 4, sl] + rows_v[r0 + 5, sl]
        s3 = rows_v[r0 + 6, sl] + rows_v[r0 + 7, sl]
        s4 = rows_v[r0 + 8, sl] + rows_v[r0 + 9, sl]
        s5 = rows_v[r0 + 10, sl] + rows_v[r0 + 11, sl]
        s6 = rows_v[r0 + 12, sl] + rows_v[r0 + 13, sl]
        s7 = rows_v[r0 + 14, sl] + rows_v[r0 + 15, sl]
        t0 = s0 + s1
        t1 = s2 + s3
        t2 = s4 + s5
        t3 = s6 + s7
        return (t0 + t1) + (t2 + t3)

    def _process(ph):
        rows_v = rows_bufs[ph]
        ids_v = ids_bufs[ph]

        def _group(g, carry):
            r0 = g * L
            idvec = ids_v[pl.ds(r0, L)]
            s_first = idvec[0]
            s_last = idvec[L - 1]

            @pl.when(s_first == s_last)
            def _uniform():
                for j in range(N_CHUNK):
                    sl = pl.ds(j * L, L)
                    total = _tree16(rows_v, r0, sl)
                    acc_v[s_first, sl] = acc_v[s_first, sl] + total
                cnt_v[s_first, :] = cnt_v[s_first, :] + (one16 * 16.0)

            @pl.when(s_first != s_last)
            def _mixed():
                for r in range(L):
                    seg = idvec[r]
                    for j in range(N_CHUNK):
                        sl = pl.ds(j * L, L)
                        acc_v[seg, sl] = acc_v[seg, sl] + rows_v[r0 + r, sl]
                    cnt_v[seg, :] = cnt_v[seg, :] + one16

            return carry

        ida = ids_v[pl.ds(0, L)]
        idb = ids_v[pl.ds(BLOCK - L, L)]
        b_first = ida[0]
        b_last = idb[L - 1]

        @pl.when(b_first == b_last)
        def _block_uniform():
            def _gsum(g, run):
                r0 = g * L
                return tuple(
                    run[j] + _tree16(rows_v, r0, pl.ds(j * L, L))
                    for j in range(N_CHUNK)
                )

            run = lax.fori_loop(
                0, BLOCK // L, _gsum, tuple(zero16 for _ in range(N_CHUNK))
            )
            for j in range(N_CHUNK):
                sl = pl.ds(j * L, L)
                acc_v[b_first, sl] = acc_v[b_first, sl] + run[j]
            cnt_v[b_first, :] = cnt_v[b_first, :] + (one16 * float(BLOCK))

        @pl.when(b_first != b_last)
        def _block_mixed():
            lax.fori_loop(0, BLOCK // L, _group, 0)

    for ph in range(NBUF):
        _issue(ph, ph)

    def _zero_acc(i, carry):
        s = i // N_CHUNK
        j = i % N_CHUNK
        acc_v[s, pl.ds(j * L, L)] = zero16
        return carry

    lax.fori_loop(0, N_SEG * N_CHUNK, _zero_acc, 0)

    def _zero_cnt(s, carry):
        cnt_v[s, :] = zero16
        return carry

    lax.fori_loop(0, N_SEG, _zero_cnt, 0)

    def _round(kp, carry):
        for ph in range(NBUF):
            k = kp * NBUF + ph
            blk = wid + k * NW

            @pl.when(blk < n_blocks)
            def _():
                _wait(ph)
                _process(ph)
                _issue(k + NBUF, ph)

        return carry

    lax.fori_loop(0, n_rounds, _round, 0)

    pltpu.sync_copy(acc_v, pa_hbm.at[:, wid, :])
    pltpu.sync_copy(cnt_v, pc_hbm.at[:, wid, :])


def _partial_tc_body(ids_ref, rows_ref, sum_ref, cnt_ref):
    i = pl.program_id(0)
    ids2d = ids_ref[0]
    iota_seg = lax.broadcasted_iota(jnp.int32, (N_SEG, RB_TC), 0)
    a = (iota_seg == jnp.broadcast_to(ids2d, (N_SEG, RB_TC))).astype(jnp.float32)
    psum = jnp.dot(a, rows_ref[...], preferred_element_type=jnp.float32)
    pcnt = jnp.broadcast_to(jnp.sum(a, axis=1)[:, None], (N_SEG, D))

    @pl.when(i == 0)
    def _init():
        sum_ref[...] = psum
        cnt_ref[...] = pcnt

    @pl.when(i != 0)
    def _acc():
        sum_ref[...] += psum
        cnt_ref[...] += pcnt


def _merge_tc_body(pa_ref, pc_ref, tsum_ref, tcnt_ref, out_ref):
    sums = jnp.sum(pa_ref[...], axis=1) + tsum_ref[...]
    cnts = jnp.sum(pc_ref[...], axis=1)[:, 0:1] + tcnt_ref[:, 0:1]
    denom = jnp.maximum(cnts, 1.0)
    out_ref[...] = sums / denom


@jax.jit
def kernel(node_features, batch):
    mesh = plsc.VectorSubcoreMesh(
        core_axis_name="c", subcore_axis_name="s", num_cores=NC, num_subcores=NS
    )

    partial_fn = pl.kernel(
        _partial_body,
        out_type=(
            jax.ShapeDtypeStruct((N_SEG, NW, D), jnp.float32),
            jax.ShapeDtypeStruct((N_SEG, NW, L), jnp.float32),
        ),
        mesh=mesh,
        scratch_types=(
            pltpu.VMEM((BLOCK, D), jnp.float32),
            pltpu.VMEM((BLOCK, D), jnp.float32),
            pltpu.VMEM((BLOCK,), jnp.int32),
            pltpu.VMEM((BLOCK,), jnp.int32),
            pltpu.VMEM((N_SEG, D), jnp.float32),
            pltpu.VMEM((N_SEG, L), jnp.float32),
            pltpu.SemaphoreType.DMA,
            pltpu.SemaphoreType.DMA,
        ),
    )
    pa, pc = partial_fn(node_features, batch)

    n_rows = node_features.shape[0]
    n_tc = n_rows - SC_ROWS
    n_tb = n_tc // RB_TC
    tb0 = SC_ROWS // RB_TC
    ids_tc = batch.reshape(n_rows // RB_TC, 1, RB_TC)
    rows_tc = node_features
    tsum, tcnt = pl.pallas_call(
        _partial_tc_body,
        grid=(n_tb,),
        in_specs=[
            pl.BlockSpec((1, 1, RB_TC), lambda i: (tb0 + i, 0, 0)),
            pl.BlockSpec((RB_TC, D), lambda i: (tb0 + i, 0)),
        ],
        out_specs=[
            pl.BlockSpec((N_SEG, D), lambda i: (0, 0)),
            pl.BlockSpec((N_SEG, D), lambda i: (0, 0)),
        ],
        out_shape=(
            jax.ShapeDtypeStruct((N_SEG, D), jnp.float32),
            jax.ShapeDtypeStruct((N_SEG, D), jnp.float32),
        ),
    )(ids_tc, rows_tc)

    return pl.pallas_call(
        _merge_tc_body,
        out_shape=jax.ShapeDtypeStruct((N_SEG, D), jnp.float32),
    )(pa, pc, tsum, tcnt)


# speedup vs baseline: 1.3593x; 1.0112x over previous
import functools

import jax
import jax.numpy as jnp
from jax import lax
from jax.experimental import pallas as pl
from jax.experimental.pallas import tpu as pltpu
from jax.experimental.pallas import tpu_sc as plsc

N_SEG = 64
D = 128
L = 16
NC = 2
NS = 16
NW = NC * NS
BLOCK = 80
NBUF = 4
N_CHUNK = D // L


SC_ROWS = 50000
RB_TC = 2000


def _partial_body(
    nf_hbm, ids_hbm, pa_hbm, pc_hbm,
    rows0_v, rows1_v, rows2_v, rows3_v, ids0_v, ids1_v, ids2_v, ids3_v,
    acc_v, cnt_v, sem0, sem1, sem2, sem3,
):
    n_blocks = SC_ROWS // BLOCK
    max_k = (n_blocks + NW - 1) // NW
    n_rounds = (max_k + NBUF - 1) // NBUF

    cid = lax.axis_index("c")
    sid = lax.axis_index("s")
    wid = sid * NC + cid

    zero16 = jnp.zeros((L,), jnp.float32)
    one16 = jnp.ones((L,), jnp.float32)
    sems = (sem0, sem1, sem2, sem3)
    rows_bufs = (rows0_v, rows1_v, rows2_v, rows3_v)
    ids_bufs = (ids0_v, ids1_v, ids2_v, ids3_v)

    def _issue(k, ph):
        blk = wid + k * NW

        @pl.when(blk < n_blocks)
        def _():
            base = blk * BLOCK
            pltpu.async_copy(
                nf_hbm.at[pl.ds(base, BLOCK), :], rows_bufs[ph], sems[ph]
            )
            pltpu.async_copy(ids_hbm.at[pl.ds(base, BLOCK)], ids_bufs[ph], sems[ph])

    def _wait(ph):
        pltpu.make_async_copy(
            nf_hbm.at[pl.ds(0, BLOCK), :], rows_bufs[ph], sems[ph]
        ).wait()
        pltpu.make_async_copy(
            ids_hbm.at[pl.ds(0, BLOCK)], ids_bufs[ph], sems[ph]
        ).wait()

    def _tree16(rows_v, r0, sl):
        s0 = rows_v[r0 + 0, sl] + rows_v[r0 + 1, sl]
        s1 = rows_v[r0 + 2, sl] + rows_v[r0 + 3, sl]
        s2 = rows_v[r0 + 4, sl] + rows_v[r0 + 5, sl]
        s3 = rows_v[r0 + 6, sl] + rows_v[r0 + 7, sl]
        s4 = rows_v[r0 + 8, sl] + rows_v[r0 + 9, sl]
        s5 = rows_v[r0 + 10, sl] + rows_v[r0 + 11, sl]
        s6 = rows_v[r0 + 12, sl] + rows_v[r0 + 13, sl]
        s7 = rows_v[r0 + 14, sl] + rows_v[r0 + 15, sl]
        t0 = s0 + s1
        t1 = s2 + s3
        t2 = s4 + s5
        t3 = s6 + s7
        return (t0 + t1) + (t2 + t3)

    def _process(ph):
        rows_v = rows_bufs[ph]
        ids_v = ids_bufs[ph]

        def _group(g, carry):
            r0 = g * L
            idvec = ids_v[pl.ds(r0, L)]
            s_first = idvec[0]
            s_last = idvec[L - 1]

            @pl.when(s_first == s_last)
            def _uniform():
                for j in range(N_CHUNK):
                    sl = pl.ds(j * L, L)
                    total = _tree16(rows_v, r0, sl)
                    acc_v[s_first, sl] = acc_v[s_first, sl] + total
                cnt_v[s_first, :] = cnt_v[s_first, :] + (one16 * 16.0)

            @pl.when(s_first != s_last)
            def _mixed():
                for r in range(L):
                    seg = idvec[r]
                    for j in range(N_CHUNK):
                        sl = pl.ds(j * L, L)
                        acc_v[seg, sl] = acc_v[seg, sl] + rows_v[r0 + r, sl]
                    cnt_v[seg, :] = cnt_v[seg, :] + one16

            return carry

        ida = ids_v[pl.ds(0, L)]
        idb = ids_v[pl.ds(BLOCK - L, L)]
        b_first = ida[0]
        b_last = idb[L - 1]

        @pl.when(b_first == b_last)
        def _block_uniform():
            def _gsum(g, run):
                r0 = g * L
                return tuple(
                    run[j] + _tree16(rows_v, r0, pl.ds(j * L, L))
                    for j in range(N_CHUNK)
                )

            run = lax.fori_loop(
                0, BLOCK // L, _gsum, tuple(zero16 for _ in range(N_CHUNK))
            )
            for j in range(N_CHUNK):
                sl = pl.ds(j * L, L)
                acc_v[b_first, sl] = acc_v[b_first, sl] + run[j]
            cnt_v[b_first, :] = cnt_v[b_first, :] + (one16 * float(BLOCK))

        @pl.when(b_first != b_last)
        def _block_mixed():
            lax.fori_loop(0, BLOCK // L, _group, 0)

    for ph in range(NBUF):
        _issue(ph, ph)

    def _zero_seg(i, carry):
        for j in range(N_CHUNK):
            acc_v[i, pl.ds(j * L, L)] = zero16
        cnt_v[i, :] = zero16
        return carry

    lax.fori_loop(0, N_SEG, _zero_seg, 0)

    def _round(kp, carry):
        for ph in range(NBUF):
            k = kp * NBUF + ph
            blk = wid + k * NW

            @pl.when(blk < n_blocks)
            def _():
                _wait(ph)
                _process(ph)
                _issue(k + NBUF, ph)

        return carry

    lax.fori_loop(0, n_rounds, _round, 0)

    pltpu.sync_copy(acc_v, pa_hbm.at[:, wid, :])
    pltpu.sync_copy(cnt_v, pc_hbm.at[:, wid, :])


def _partial_tc_body(ids_ref, rows_ref, sum_ref, cnt_ref):
    i = pl.program_id(0)
    ids2d = ids_ref[0]
    iota_seg = lax.broadcasted_iota(jnp.int32, (N_SEG, RB_TC), 0)
    a = (iota_seg == jnp.broadcast_to(ids2d, (N_SEG, RB_TC))).astype(jnp.float32)
    psum = jnp.dot(a, rows_ref[...], preferred_element_type=jnp.float32)
    pcnt = jnp.broadcast_to(jnp.sum(a, axis=1)[:, None], (N_SEG, D))

    @pl.when(i == 0)
    def _init():
        sum_ref[...] = psum
        cnt_ref[...] = pcnt

    @pl.when(i != 0)
    def _acc():
        sum_ref[...] += psum
        cnt_ref[...] += pcnt


def _merge_tc_body(pa_ref, pc_ref, tsum_ref, tcnt_ref, out_ref):
    sums = jnp.sum(pa_ref[...], axis=1) + tsum_ref[...]
    cnts = jnp.sum(pc_ref[...], axis=1)[:, 0:1] + tcnt_ref[:, 0:1]
    denom = jnp.maximum(cnts, 1.0)
    out_ref[...] = sums / denom


@jax.jit
def kernel(node_features, batch):
    mesh = plsc.VectorSubcoreMesh(
        core_axis_name="c", subcore_axis_name="s", num_cores=NC, num_subcores=NS
    )

    partial_fn = pl.kernel(
        _partial_body,
        out_type=(
            jax.ShapeDtypeStruct((N_SEG, NW, D), jnp.float32),
            jax.ShapeDtypeStruct((N_SEG, NW, L), jnp.float32),
        ),
        mesh=mesh,
        scratch_types=(
            pltpu.VMEM((BLOCK, D), jnp.float32),
            pltpu.VMEM((BLOCK, D), jnp.float32),
            pltpu.VMEM((BLOCK, D), jnp.float32),
            pltpu.VMEM((BLOCK, D), jnp.float32),
            pltpu.VMEM((BLOCK,), jnp.int32),
            pltpu.VMEM((BLOCK,), jnp.int32),
            pltpu.VMEM((BLOCK,), jnp.int32),
            pltpu.VMEM((BLOCK,), jnp.int32),
            pltpu.VMEM((N_SEG, D), jnp.float32),
            pltpu.VMEM((N_SEG, L), jnp.float32),
            pltpu.SemaphoreType.DMA,
            pltpu.SemaphoreType.DMA,
            pltpu.SemaphoreType.DMA,
            pltpu.SemaphoreType.DMA,
        ),
    )
    pa, pc = partial_fn(node_features, batch)

    n_rows = node_features.shape[0]
    n_tc = n_rows - SC_ROWS
    n_tb = n_tc // RB_TC
    tb0 = SC_ROWS // RB_TC
    ids_tc = batch.reshape(n_rows // RB_TC, 1, RB_TC)
    rows_tc = node_features
    tsum, tcnt = pl.pallas_call(
        _partial_tc_body,
        grid=(n_tb,),
        in_specs=[
            pl.BlockSpec((1, 1, RB_TC), lambda i: (tb0 + i, 0, 0)),
            pl.BlockSpec((RB_TC, D), lambda i: (tb0 + i, 0)),
        ],
        out_specs=[
            pl.BlockSpec((N_SEG, D), lambda i: (0, 0)),
            pl.BlockSpec((N_SEG, D), lambda i: (0, 0)),
        ],
        out_shape=(
            jax.ShapeDtypeStruct((N_SEG, D), jnp.float32),
            jax.ShapeDtypeStruct((N_SEG, D), jnp.float32),
        ),
    )(ids_tc, rows_tc)

    return pl.pallas_call(
        _merge_tc_body,
        out_shape=jax.ShapeDtypeStruct((N_SEG, D), jnp.float32),
    )(pa, pc, tsum, tcnt)


# speedup vs baseline: 1.3635x; 1.0031x over previous
import functools

import jax
import jax.numpy as jnp
from jax import lax
from jax.experimental import pallas as pl
from jax.experimental.pallas import tpu as pltpu
from jax.experimental.pallas import tpu_sc as plsc

N_SEG = 64
D = 128
L = 16
NC = 2
NS = 16
NW = NC * NS
BLOCK = 80
NBUF = 4
N_CHUNK = D // L


SC_ROWS = 46000
RB_TC = 2000


def _partial_body(
    nf_hbm, ids_hbm, pa_hbm, pc_hbm,
    rows0_v, rows1_v, rows2_v, rows3_v, ids0_v, ids1_v, ids2_v, ids3_v,
    acc_v, cnt_v, sem0, sem1, sem2, sem3,
):
    n_blocks = SC_ROWS // BLOCK
    max_k = (n_blocks + NW - 1) // NW
    n_rounds = (max_k + NBUF - 1) // NBUF

    cid = lax.axis_index("c")
    sid = lax.axis_index("s")
    wid = sid * NC + cid

    zero16 = jnp.zeros((L,), jnp.float32)
    one16 = jnp.ones((L,), jnp.float32)
    sems = (sem0, sem1, sem2, sem3)
    rows_bufs = (rows0_v, rows1_v, rows2_v, rows3_v)
    ids_bufs = (ids0_v, ids1_v, ids2_v, ids3_v)

    def _issue(k, ph):
        blk = wid + k * NW

        @pl.when(blk < n_blocks)
        def _():
            base = blk * BLOCK
            pltpu.async_copy(
                nf_hbm.at[pl.ds(base, BLOCK), :], rows_bufs[ph], sems[ph]
            )
            pltpu.async_copy(ids_hbm.at[pl.ds(base, BLOCK)], ids_bufs[ph], sems[ph])

    def _wait(ph):
        pltpu.make_async_copy(
            nf_hbm.at[pl.ds(0, BLOCK), :], rows_bufs[ph], sems[ph]
        ).wait()
        pltpu.make_async_copy(
            ids_hbm.at[pl.ds(0, BLOCK)], ids_bufs[ph], sems[ph]
        ).wait()

    def _tree16(rows_v, r0, sl):
        s0 = rows_v[r0 + 0, sl] + rows_v[r0 + 1, sl]
        s1 = rows_v[r0 + 2, sl] + rows_v[r0 + 3, sl]
        s2 = rows_v[r0 + 4, sl] + rows_v[r0 + 5, sl]
        s3 = rows_v[r0 + 6, sl] + rows_v[r0 + 7, sl]
        s4 = rows_v[r0 + 8, sl] + rows_v[r0 + 9, sl]
        s5 = rows_v[r0 + 10, sl] + rows_v[r0 + 11, sl]
        s6 = rows_v[r0 + 12, sl] + rows_v[r0 + 13, sl]
        s7 = rows_v[r0 + 14, sl] + rows_v[r0 + 15, sl]
        t0 = s0 + s1
        t1 = s2 + s3
        t2 = s4 + s5
        t3 = s6 + s7
        return (t0 + t1) + (t2 + t3)

    def _process(ph):
        rows_v = rows_bufs[ph]
        ids_v = ids_bufs[ph]

        def _group(g, carry):
            r0 = g * L
            idvec = ids_v[pl.ds(r0, L)]
            s_first = idvec[0]
            s_last = idvec[L - 1]

            @pl.when(s_first == s_last)
            def _uniform():
                for j in range(N_CHUNK):
                    sl = pl.ds(j * L, L)
                    total = _tree16(rows_v, r0, sl)
                    acc_v[s_first, sl] = acc_v[s_first, sl] + total
                cnt_v[s_first, :] = cnt_v[s_first, :] + (one16 * 16.0)

            @pl.when(s_first != s_last)
            def _mixed():
                for r in range(L):
                    seg = idvec[r]
                    for j in range(N_CHUNK):
                        sl = pl.ds(j * L, L)
                        acc_v[seg, sl] = acc_v[seg, sl] + rows_v[r0 + r, sl]
                    cnt_v[seg, :] = cnt_v[seg, :] + one16

            return carry

        ida = ids_v[pl.ds(0, L)]
        idb = ids_v[pl.ds(BLOCK - L, L)]
        b_first = ida[0]
        b_last = idb[L - 1]

        @pl.when(b_first == b_last)
        def _block_uniform():
            def _gsum(g, run):
                r0 = g * L
                return tuple(
                    run[j] + _tree16(rows_v, r0, pl.ds(j * L, L))
                    for j in range(N_CHUNK)
                )

            run = lax.fori_loop(
                0, BLOCK // L, _gsum, tuple(zero16 for _ in range(N_CHUNK))
            )
            for j in range(N_CHUNK):
                sl = pl.ds(j * L, L)
                acc_v[b_first, sl] = acc_v[b_first, sl] + run[j]
            cnt_v[b_first, :] = cnt_v[b_first, :] + (one16 * float(BLOCK))

        @pl.when(b_first != b_last)
        def _block_mixed():
            lax.fori_loop(0, BLOCK // L, _group, 0)

    for ph in range(NBUF):
        _issue(ph, ph)

    def _zero_seg(i, carry):
        for j in range(N_CHUNK):
            acc_v[i, pl.ds(j * L, L)] = zero16
        cnt_v[i, :] = zero16
        return carry

    lax.fori_loop(0, N_SEG, _zero_seg, 0)

    def _round(kp, carry):
        for ph in range(NBUF):
            k = kp * NBUF + ph
            blk = wid + k * NW

            @pl.when(blk < n_blocks)
            def _():
                _wait(ph)
                _process(ph)
                _issue(k + NBUF, ph)

        return carry

    lax.fori_loop(0, n_rounds, _round, 0)

    pltpu.sync_copy(acc_v, pa_hbm.at[:, wid, :])
    pltpu.sync_copy(cnt_v, pc_hbm.at[:, wid, :])


def _partial_tc_body(ids_ref, rows_ref, sum_ref, cnt_ref):
    i = pl.program_id(0)
    ids2d = ids_ref[0]
    iota_seg = lax.broadcasted_iota(jnp.int32, (N_SEG, RB_TC), 0)
    onehot = iota_seg == jnp.broadcast_to(ids2d, (N_SEG, RB_TC))
    a_bf = onehot.astype(jnp.bfloat16)
    rows_bf = rows_ref[...].astype(jnp.bfloat16)
    a = onehot.astype(jnp.float32)
    psum = jnp.dot(a_bf, rows_bf, preferred_element_type=jnp.float32)
    pcnt = jnp.broadcast_to(jnp.sum(a, axis=1)[:, None], (N_SEG, D))

    @pl.when(i == 0)
    def _init():
        sum_ref[...] = psum
        cnt_ref[...] = pcnt

    @pl.when(i != 0)
    def _acc():
        sum_ref[...] += psum
        cnt_ref[...] += pcnt


def _merge_tc_body(pa_ref, pc_ref, tsum_ref, tcnt_ref, out_ref):
    sums = jnp.sum(pa_ref[...], axis=1) + tsum_ref[...]
    cnts = jnp.sum(pc_ref[...], axis=1)[:, 0:1] + tcnt_ref[:, 0:1]
    denom = jnp.maximum(cnts, 1.0)
    out_ref[...] = sums / denom


@jax.jit
def kernel(node_features, batch):
    mesh = plsc.VectorSubcoreMesh(
        core_axis_name="c", subcore_axis_name="s", num_cores=NC, num_subcores=NS
    )

    partial_fn = pl.kernel(
        _partial_body,
        out_type=(
            jax.ShapeDtypeStruct((N_SEG, NW, D), jnp.float32),
            jax.ShapeDtypeStruct((N_SEG, NW, L), jnp.float32),
        ),
        mesh=mesh,
        scratch_types=(
            pltpu.VMEM((BLOCK, D), jnp.float32),
            pltpu.VMEM((BLOCK, D), jnp.float32),
            pltpu.VMEM((BLOCK, D), jnp.float32),
            pltpu.VMEM((BLOCK, D), jnp.float32),
            pltpu.VMEM((BLOCK,), jnp.int32),
            pltpu.VMEM((BLOCK,), jnp.int32),
            pltpu.VMEM((BLOCK,), jnp.int32),
            pltpu.VMEM((BLOCK,), jnp.int32),
            pltpu.VMEM((N_SEG, D), jnp.float32),
            pltpu.VMEM((N_SEG, L), jnp.float32),
            pltpu.SemaphoreType.DMA,
            pltpu.SemaphoreType.DMA,
            pltpu.SemaphoreType.DMA,
            pltpu.SemaphoreType.DMA,
        ),
    )
    pa, pc = partial_fn(node_features, batch)

    n_rows = node_features.shape[0]
    n_tc = n_rows - SC_ROWS
    n_tb = n_tc // RB_TC
    tb0 = SC_ROWS // RB_TC
    ids_tc = batch.reshape(n_rows // RB_TC, 1, RB_TC)
    rows_tc = node_features
    tsum, tcnt = pl.pallas_call(
        _partial_tc_body,
        grid=(n_tb,),
        in_specs=[
            pl.BlockSpec((1, 1, RB_TC), lambda i: (tb0 + i, 0, 0)),
            pl.BlockSpec((RB_TC, D), lambda i: (tb0 + i, 0)),
        ],
        out_specs=[
            pl.BlockSpec((N_SEG, D), lambda i: (0, 0)),
            pl.BlockSpec((N_SEG, D), lambda i: (0, 0)),
        ],
        out_shape=(
            jax.ShapeDtypeStruct((N_SEG, D), jnp.float32),
            jax.ShapeDtypeStruct((N_SEG, D), jnp.float32),
        ),
    )(ids_tc, rows_tc)

    return pl.pallas_call(
        _merge_tc_body,
        out_shape=jax.ShapeDtypeStruct((N_SEG, D), jnp.float32),
    )(pa, pc, tsum, tcnt)


# speedup vs baseline: 1.3878x; 1.0178x over previous
import functools

import jax
import jax.numpy as jnp
from jax import lax
from jax.experimental import pallas as pl
from jax.experimental.pallas import tpu as pltpu
from jax.experimental.pallas import tpu_sc as plsc

N_SEG = 64
D = 128
L = 16
NC = 2
NS = 16
NW = NC * NS
BLOCK = 80
NBUF = 4
N_CHUNK = D // L


SC_ROWS = 44000
RB_TC = 4000


def _partial_body(
    nf_hbm, ids_hbm, pa_hbm, pc_hbm,
    rows0_v, rows1_v, rows2_v, rows3_v, ids0_v, ids1_v, ids2_v, ids3_v,
    acc_v, cnt_v, sem0, sem1, sem2, sem3,
):
    n_blocks = SC_ROWS // BLOCK
    max_k = (n_blocks + NW - 1) // NW
    n_rounds = (max_k + NBUF - 1) // NBUF

    cid = lax.axis_index("c")
    sid = lax.axis_index("s")
    wid = sid * NC + cid

    zero16 = jnp.zeros((L,), jnp.float32)
    one16 = jnp.ones((L,), jnp.float32)
    sems = (sem0, sem1, sem2, sem3)
    rows_bufs = (rows0_v, rows1_v, rows2_v, rows3_v)
    ids_bufs = (ids0_v, ids1_v, ids2_v, ids3_v)

    def _issue(k, ph):
        blk = wid + k * NW

        @pl.when(blk < n_blocks)
        def _():
            base = blk * BLOCK
            pltpu.async_copy(
                nf_hbm.at[pl.ds(base, BLOCK), :], rows_bufs[ph], sems[ph]
            )
            pltpu.async_copy(ids_hbm.at[pl.ds(base, BLOCK)], ids_bufs[ph], sems[ph])

    def _wait(ph):
        pltpu.make_async_copy(
            nf_hbm.at[pl.ds(0, BLOCK), :], rows_bufs[ph], sems[ph]
        ).wait()
        pltpu.make_async_copy(
            ids_hbm.at[pl.ds(0, BLOCK)], ids_bufs[ph], sems[ph]
        ).wait()

    def _tree16(rows_v, r0, sl):
        s0 = rows_v[r0 + 0, sl] + rows_v[r0 + 1, sl]
        s1 = rows_v[r0 + 2, sl] + rows_v[r0 + 3, sl]
        s2 = rows_v[r0 + 4, sl] + rows_v[r0 + 5, sl]
        s3 = rows_v[r0 + 6, sl] + rows_v[r0 + 7, sl]
        s4 = rows_v[r0 + 8, sl] + rows_v[r0 + 9, sl]
        s5 = rows_v[r0 + 10, sl] + rows_v[r0 + 11, sl]
        s6 = rows_v[r0 + 12, sl] + rows_v[r0 + 13, sl]
        s7 = rows_v[r0 + 14, sl] + rows_v[r0 + 15, sl]
        t0 = s0 + s1
        t1 = s2 + s3
        t2 = s4 + s5
        t3 = s6 + s7
        return (t0 + t1) + (t2 + t3)

    def _process(ph):
        rows_v = rows_bufs[ph]
        ids_v = ids_bufs[ph]

        def _group(g, carry):
            r0 = g * L
            idvec = ids_v[pl.ds(r0, L)]
            s_first = idvec[0]
            s_last = idvec[L - 1]

            @pl.when(s_first == s_last)
            def _uniform():
                for j in range(N_CHUNK):
                    sl = pl.ds(j * L, L)
                    total = _tree16(rows_v, r0, sl)
                    acc_v[s_first, sl] = acc_v[s_first, sl] + total
                cnt_v[s_first, :] = cnt_v[s_first, :] + (one16 * 16.0)

            @pl.when(s_first != s_last)
            def _mixed():
                for r in range(L):
                    seg = idvec[r]
                    for j in range(N_CHUNK):
                        sl = pl.ds(j * L, L)
                        acc_v[seg, sl] = acc_v[seg, sl] + rows_v[r0 + r, sl]
                    cnt_v[seg, :] = cnt_v[seg, :] + one16

            return carry

        ida = ids_v[pl.ds(0, L)]
        idb = ids_v[pl.ds(BLOCK - L, L)]
        b_first = ida[0]
        b_last = idb[L - 1]

        @pl.when(b_first == b_last)
        def _block_uniform():
            def _gsum(g, run):
                r0 = g * L
                return tuple(
                    run[j] + _tree16(rows_v, r0, pl.ds(j * L, L))
                    for j in range(N_CHUNK)
                )

            run = lax.fori_loop(
                0, BLOCK // L, _gsum, tuple(zero16 for _ in range(N_CHUNK))
            )
            for j in range(N_CHUNK):
                sl = pl.ds(j * L, L)
                acc_v[b_first, sl] = acc_v[b_first, sl] + run[j]
            cnt_v[b_first, :] = cnt_v[b_first, :] + (one16 * float(BLOCK))

        @pl.when(b_first != b_last)
        def _block_mixed():
            lax.fori_loop(0, BLOCK // L, _group, 0)

    for ph in range(NBUF):
        _issue(ph, ph)

    def _zero_seg(i, carry):
        for j in range(N_CHUNK):
            acc_v[i, pl.ds(j * L, L)] = zero16
        cnt_v[i, :] = zero16
        return carry

    lax.fori_loop(0, N_SEG, _zero_seg, 0)

    def _round(kp, carry):
        for ph in range(NBUF):
            k = kp * NBUF + ph
            blk = wid + k * NW

            @pl.when(blk < n_blocks)
            def _():
                _wait(ph)
                _process(ph)
                _issue(k + NBUF, ph)

        return carry

    lax.fori_loop(0, n_rounds, _round, 0)

    pltpu.sync_copy(acc_v, pa_hbm.at[:, wid, :])
    pltpu.sync_copy(cnt_v, pc_hbm.at[:, wid, :])


def _partial_tc_body(ids_ref, rows_ref, sum_ref, cnt_ref):
    i = pl.program_id(0)
    ids2d = ids_ref[0]
    iota_seg = lax.broadcasted_iota(jnp.int32, (N_SEG, RB_TC), 0)
    onehot = iota_seg == jnp.broadcast_to(ids2d, (N_SEG, RB_TC))
    a_bf = onehot.astype(jnp.bfloat16)
    rows_bf = rows_ref[...].astype(jnp.bfloat16)
    psum = jnp.dot(a_bf, rows_bf, preferred_element_type=jnp.float32)
    ones_bf = jnp.ones((RB_TC, D), jnp.bfloat16)
    pcnt = jnp.dot(a_bf, ones_bf, preferred_element_type=jnp.float32)

    @pl.when(i == 0)
    def _init():
        sum_ref[...] = psum
        cnt_ref[...] = pcnt

    @pl.when(i != 0)
    def _acc():
        sum_ref[...] += psum
        cnt_ref[...] += pcnt


def _merge_tc_body(pa_ref, pc_ref, tsum_ref, tcnt_ref, out_ref):
    sums = jnp.sum(pa_ref[...], axis=1) + tsum_ref[...]
    cnts = jnp.sum(pc_ref[...], axis=1)[:, 0:1] + tcnt_ref[:, 0:1]
    denom = jnp.maximum(cnts, 1.0)
    out_ref[...] = sums / denom


@jax.jit
def kernel(node_features, batch):
    mesh = plsc.VectorSubcoreMesh(
        core_axis_name="c", subcore_axis_name="s", num_cores=NC, num_subcores=NS
    )

    partial_fn = pl.kernel(
        _partial_body,
        out_type=(
            jax.ShapeDtypeStruct((N_SEG, NW, D), jnp.float32),
            jax.ShapeDtypeStruct((N_SEG, NW, L), jnp.float32),
        ),
        mesh=mesh,
        scratch_types=(
            pltpu.VMEM((BLOCK, D), jnp.float32),
            pltpu.VMEM((BLOCK, D), jnp.float32),
            pltpu.VMEM((BLOCK, D), jnp.float32),
            pltpu.VMEM((BLOCK, D), jnp.float32),
            pltpu.VMEM((BLOCK,), jnp.int32),
            pltpu.VMEM((BLOCK,), jnp.int32),
            pltpu.VMEM((BLOCK,), jnp.int32),
            pltpu.VMEM((BLOCK,), jnp.int32),
            pltpu.VMEM((N_SEG, D), jnp.float32),
            pltpu.VMEM((N_SEG, L), jnp.float32),
            pltpu.SemaphoreType.DMA,
            pltpu.SemaphoreType.DMA,
            pltpu.SemaphoreType.DMA,
            pltpu.SemaphoreType.DMA,
        ),
    )
    pa, pc = partial_fn(node_features, batch)

    n_rows = node_features.shape[0]
    n_tc = n_rows - SC_ROWS
    n_tb = n_tc // RB_TC
    tb0 = SC_ROWS // RB_TC
    ids_tc = batch.reshape(n_rows // RB_TC, 1, RB_TC)
    rows_tc = node_features
    tsum, tcnt = pl.pallas_call(
        _partial_tc_body,
        grid=(n_tb,),
        in_specs=[
            pl.BlockSpec((1, 1, RB_TC), lambda i: (tb0 + i, 0, 0)),
            pl.BlockSpec((RB_TC, D), lambda i: (tb0 + i, 0)),
        ],
        out_specs=[
            pl.BlockSpec((N_SEG, D), lambda i: (0, 0)),
            pl.BlockSpec((N_SEG, D), lambda i: (0, 0)),
        ],
        out_shape=(
            jax.ShapeDtypeStruct((N_SEG, D), jnp.float32),
            jax.ShapeDtypeStruct((N_SEG, D), jnp.float32),
        ),
    )(ids_tc, rows_tc)

    return pl.pallas_call(
        _merge_tc_body,
        out_shape=jax.ShapeDtypeStruct((N_SEG, D), jnp.float32),
    )(pa, pc, tsum, tcnt)
